# Initial kernel scaffold; baseline (speedup 1.0000x reference)
import jax
import jax.numpy as jnp
from jax.experimental import pallas as pl


def kernel(x, Wg, W1, W2, W3):
    raise NotImplementedError("write your pallas kernel here")


# speedup vs baseline: 1.5560x; 1.5560x over previous
import jax
import jax.numpy as jnp
from jax.experimental import pallas as pl
from jax.experimental.pallas import tpu as pltpu

D_MODEL = 768
D_FFN = 3072
N_EXPERTS = 8
T = 2048
F_BLK = 768
N_FBLK = D_FFN // F_BLK


def _dense_body(x_ref, wg_ref, w1_ref, w3_ref, w2_ref, out_ref, aux_ref, comb_ref):
    e = pl.program_id(0)
    f = pl.program_id(1)
    xb = x_ref[...]

    @pl.when((e == 0) & (f == 0))
    def _router():
        logits = jnp.dot(xb, wg_ref[...], preferred_element_type=jnp.float32)
        mu = jnp.mean(logits, axis=1, keepdims=True)
        var = jnp.sum((logits - mu) ** 2, axis=1, keepdims=True) / (N_EXPERTS - 1)
        aux_ref[...] = jnp.mean(var).reshape(1, 1)
        m1 = jnp.max(logits, axis=1, keepdims=True)
        p = jnp.exp(logits - m1)
        probs = p / jnp.sum(p, axis=1, keepdims=True)
        iota8 = jax.lax.broadcasted_iota(jnp.int32, (T, N_EXPERTS), 1)
        w1v = jnp.max(probs, axis=1, keepdims=True)
        e1 = jnp.min(jnp.where(probs == w1v, iota8, N_EXPERTS), axis=1, keepdims=True)
        oh1 = iota8 == e1
        pm = jnp.where(oh1, -1e30, probs)
        w2v = jnp.max(pm, axis=1, keepdims=True)
        e2 = jnp.min(jnp.where(pm == w2v, iota8, N_EXPERTS), axis=1, keepdims=True)
        oh2 = iota8 == e2
        wsum = w1v + w2v
        comb_ref[...] = (jnp.where(oh1, w1v, 0.0) + jnp.where(oh2, w2v, 0.0)) / wsum

    a = jnp.dot(xb, w1_ref[0], preferred_element_type=jnp.float32)
    b = jnp.dot(xb, w3_ref[0], preferred_element_type=jnp.float32)
    h = (a * jax.nn.sigmoid(a)) * b
    part = jnp.dot(h, w2_ref[0], preferred_element_type=jnp.float32)
    iota8 = jax.lax.broadcasted_iota(jnp.int32, (T, N_EXPERTS), 1)
    c = jnp.sum(jnp.where(iota8 == e, comb_ref[...], 0.0), axis=1, keepdims=True)

    @pl.when((e == 0) & (f == 0))
    def _init():
        out_ref[...] = part * c

    @pl.when((e != 0) | (f != 0))
    def _acc():
        out_ref[...] += part * c


def kernel(x, Wg, W1, W2, W3):
    x2 = x.reshape(T, D_MODEL)
    out, aux = pl.pallas_call(
        _dense_body,
        grid=(N_EXPERTS, N_FBLK),
        in_specs=[
            pl.BlockSpec((T, D_MODEL), lambda e, f: (0, 0)),
            pl.BlockSpec((D_MODEL, N_EXPERTS), lambda e, f: (0, 0)),
            pl.BlockSpec((1, D_MODEL, F_BLK), lambda e, f: (e, 0, f)),
            pl.BlockSpec((1, D_MODEL, F_BLK), lambda e, f: (e, 0, f)),
            pl.BlockSpec((1, F_BLK, D_MODEL), lambda e, f: (e, f, 0)),
        ],
        out_specs=[
            pl.BlockSpec((T, D_MODEL), lambda e, f: (0, 0)),
            pl.BlockSpec((1, 1), lambda e, f: (0, 0)),
        ],
        out_shape=[
            jax.ShapeDtypeStruct((T, D_MODEL), jnp.float32),
            jax.ShapeDtypeStruct((1, 1), jnp.float32),
        ],
        scratch_shapes=[pltpu.VMEM((T, N_EXPERTS), jnp.float32)],
    )(x2, Wg, W1, W3, W2)
    return out.reshape(x.shape), aux.reshape(())


# speedup vs baseline: 1.7694x; 1.1371x over previous
import functools

import jax
import jax.numpy as jnp
from jax import lax
from jax.experimental import pallas as pl
from jax.experimental.pallas import tpu as pltpu
from jax.experimental.pallas import tpu_sc as plsc

D_MODEL = 768
D_FFN = 3072
N_EXPERTS = 8
T = 2048
F_BLK = 768
N_FBLK = D_FFN // F_BLK
ROWS = 256
MAX_TILES = 24
P = MAX_TILES * ROWS
NW = 32
TPW = T // NW


def _router_body(x_ref, wg_ref, pos1_ref, pos2_ref, w1n_ref, w2n_ref,
                 arre_ref, valid_ref, xblk_ref, aux_ref):
    xb = x_ref[...]
    logits = jnp.dot(xb, wg_ref[...], preferred_element_type=jnp.float32)
    mu = jnp.mean(logits, axis=1, keepdims=True)
    var = jnp.sum((logits - mu) ** 2, axis=1, keepdims=True) / (N_EXPERTS - 1)
    aux_ref[...] = jnp.mean(var).reshape(1, 1)

    m1 = jnp.max(logits, axis=1, keepdims=True)
    p = jnp.exp(logits - m1)
    probs = p / jnp.sum(p, axis=1, keepdims=True)
    iota8 = lax.broadcasted_iota(jnp.int32, (T, N_EXPERTS), 1)
    w1v = jnp.max(probs, axis=1, keepdims=True)
    e1 = jnp.min(jnp.where(probs == w1v, iota8, N_EXPERTS), axis=1, keepdims=True)
    oh1 = iota8 == e1
    pm = jnp.where(oh1, -1e30, probs)
    w2v = jnp.max(pm, axis=1, keepdims=True)
    e2 = jnp.min(jnp.where(pm == w2v, iota8, N_EXPERTS), axis=1, keepdims=True)
    oh2 = iota8 == e2
    wsum = w1v + w2v
    w1n_ref[...] = w1v / wsum
    w2n_ref[...] = w2v / wsum

    mask = oh1.astype(jnp.float32) + oh2.astype(jnp.float32)
    ir = lax.broadcasted_iota(jnp.int32, (T, T), 0)
    ic = lax.broadcasted_iota(jnp.int32, (T, T), 1)
    lower = (ir > ic).astype(jnp.float32)
    rank = jnp.dot(lower, mask, preferred_element_type=jnp.float32)

    cnt = jnp.sum(mask, axis=0, keepdims=True)
    padded = jnp.ceil(cnt / ROWS) * ROWS
    k8 = lax.broadcasted_iota(jnp.int32, (N_EXPERTS, N_EXPERTS), 0)
    e8 = lax.broadcasted_iota(jnp.int32, (N_EXPERTS, N_EXPERTS), 1)
    lt8 = (k8 < e8).astype(jnp.float32)
    offs = jnp.dot(padded, lt8, preferred_element_type=jnp.float32)
    n_act = jnp.sum(padded) / ROWS

    pos1_ref[...] = (jnp.sum(jnp.where(oh1, rank + offs, 0.0), axis=1,
                             keepdims=True)).astype(jnp.int32)
    pos2_ref[...] = (jnp.sum(jnp.where(oh2, rank + offs, 0.0), axis=1,
                             keepdims=True)).astype(jnp.int32)

    it = lax.broadcasted_iota(jnp.int32, (MAX_TILES, N_EXPERTS), 0).astype(jnp.float32)
    ie = lax.broadcasted_iota(jnp.int32, (MAX_TILES, N_EXPERTS), 1).astype(jnp.float32)
    i_c = jnp.minimum(it, n_act - 1.0)
    start = offs / ROWS
    ntil = padded / ROWS
    ind = ((i_c >= start) & (i_c < start + ntil)).astype(jnp.float32)
    arre_ref[...] = jnp.sum(ie * ind, axis=1, keepdims=True).astype(jnp.int32)
    it1 = lax.broadcasted_iota(jnp.int32, (MAX_TILES, 1), 0).astype(jnp.float32)
    valid_ref[...] = (it1 < n_act).astype(jnp.int32)
    xblk_ref[...] = jnp.minimum(it1, n_act - 1.0).astype(jnp.int32)


def _router(x2, Wg):
    outs = pl.pallas_call(
        _router_body,
        out_shape=[
            jax.ShapeDtypeStruct((T, 1), jnp.int32),
            jax.ShapeDtypeStruct((T, 1), jnp.int32),
            jax.ShapeDtypeStruct((T, 1), jnp.float32),
            jax.ShapeDtypeStruct((T, 1), jnp.float32),
            jax.ShapeDtypeStruct((MAX_TILES, 1), jnp.int32),
            jax.ShapeDtypeStruct((MAX_TILES, 1), jnp.int32),
            jax.ShapeDtypeStruct((MAX_TILES, 1), jnp.int32),
            jax.ShapeDtypeStruct((1, 1), jnp.float32),
        ],
    )(x2, Wg)
    return outs


@functools.cache
def _get_dispatch():
    mesh = plsc.VectorSubcoreMesh(core_axis_name="c", subcore_axis_name="s")

    @functools.partial(
        pl.kernel,
        out_type=jax.ShapeDtypeStruct((P, D_MODEL), jnp.float32),
        mesh=mesh,
        scratch_types=[
            pltpu.VMEM((TPW, D_MODEL), jnp.float32),
            pltpu.VMEM((TPW,), jnp.int32),
            pltpu.VMEM((TPW,), jnp.int32),
            pltpu.SemaphoreType.DMA,
        ],
    )
    def _dispatch(x_hbm, pos_hbm, xs_hbm, buf, idx1, idx2, sem):
        wid = lax.axis_index("s") * 2 + lax.axis_index("c")
        base = wid * TPW
        pltpu.sync_copy(x_hbm.at[pl.ds(base, TPW)], buf)
        pltpu.sync_copy(pos_hbm.at[wid], idx1)
        pltpu.sync_copy(pos_hbm.at[NW + wid], idx2)
        pltpu.async_copy(buf, xs_hbm.at[idx1], sem).wait()
        pltpu.async_copy(buf, xs_hbm.at[idx2], sem).wait()

    return _dispatch


def _ffn_body(se, sv, sx, x_ref, w1_ref, w3_ref, w2_ref, y_ref):
    i = pl.program_id(0)
    f = pl.program_id(1)

    @pl.when(sv[i] == 1)
    def _():
        xb = x_ref[...]
        a = jnp.dot(xb, w1_ref[0], preferred_element_type=jnp.float32)
        b = jnp.dot(xb, w3_ref[0], preferred_element_type=jnp.float32)
        h = (a * jax.nn.sigmoid(a)) * b
        part = jnp.dot(h, w2_ref[0], preferred_element_type=jnp.float32)

        @pl.when(f == 0)
        def _():
            y_ref[...] = part

        @pl.when(f != 0)
        def _():
            y_ref[...] += part


def _ffn(arr_e, valid, xblk, xs, W1, W3, W2):
    grid_spec = pltpu.PrefetchScalarGridSpec(
        num_scalar_prefetch=3,
        grid=(MAX_TILES, N_FBLK),
        in_specs=[
            pl.BlockSpec((ROWS, D_MODEL), lambda i, f, se, sv, sx: (sx[i], 0)),
            pl.BlockSpec((1, D_MODEL, F_BLK),
                         lambda i, f, se, sv, sx: (se[i], 0, jnp.where(sv[i] == 1, f, 0))),
            pl.BlockSpec((1, D_MODEL, F_BLK),
                         lambda i, f, se, sv, sx: (se[i], 0, jnp.where(sv[i] == 1, f, 0))),
            pl.BlockSpec((1, F_BLK, D_MODEL),
                         lambda i, f, se, sv, sx: (se[i], jnp.where(sv[i] == 1, f, 0), 0)),
        ],
        out_specs=pl.BlockSpec((ROWS, D_MODEL), lambda i, f, se, sv, sx: (sx[i], 0)),
    )
    return pl.pallas_call(
        _ffn_body,
        grid_spec=grid_spec,
        out_shape=jax.ShapeDtypeStruct((P, D_MODEL), jnp.float32),
    )(arr_e, valid, xblk, xs, W1, W3, W2)


@functools.cache
def _get_combine():
    mesh = plsc.VectorSubcoreMesh(core_axis_name="c", subcore_axis_name="s")

    @functools.partial(
        pl.kernel,
        out_type=jax.ShapeDtypeStruct((T, D_MODEL), jnp.float32),
        mesh=mesh,
        scratch_types=[
            pltpu.VMEM((TPW, D_MODEL), jnp.float32),
            pltpu.VMEM((TPW, D_MODEL), jnp.float32),
            pltpu.VMEM((TPW,), jnp.int32),
            pltpu.VMEM((TPW,), jnp.int32),
            pltpu.VMEM((TPW,), jnp.float32),
            pltpu.VMEM((TPW,), jnp.float32),
            pltpu.SemaphoreType.DMA,
        ],
    )
    def _combine(y_hbm, pos_hbm, w_hbm, out_hbm, b1, b2, idx1, idx2, wv1, wv2, sem):
        wid = lax.axis_index("s") * 2 + lax.axis_index("c")
        base = wid * TPW
        pltpu.sync_copy(pos_hbm.at[wid], idx1)
        pltpu.sync_copy(pos_hbm.at[NW + wid], idx2)
        pltpu.sync_copy(w_hbm.at[wid], wv1)
        pltpu.sync_copy(w_hbm.at[NW + wid], wv2)
        pltpu.async_copy(y_hbm.at[idx1], b1, sem).wait()
        pltpu.async_copy(y_hbm.at[idx2], b2, sem).wait()

        def body(g, carry):
            wvec1 = wv1[pl.ds(g * 16, 16)]
            wvec2 = wv2[pl.ds(g * 16, 16)]
            for lane in range(16):
                t = g * 16 + lane
                ws1 = wvec1[lane]
                ws2 = wvec2[lane]
                for j in range(D_MODEL // 16):
                    s = pl.ds(j * 16, 16)
                    b1[t, s] = ws1 * b1[t, s] + ws2 * b2[t, s]
            return carry

        lax.fori_loop(0, TPW // 16, body, 0)
        pltpu.sync_copy(b1, out_hbm.at[pl.ds(base, TPW)])

    return _combine


def kernel(x, Wg, W1, W2, W3):
    x2 = x.reshape(T, D_MODEL)
    pos1, pos2, w1n, w2n, arr_e, valid, xblk, aux = _router(x2, Wg)
    pos_all = jnp.concatenate(
        [pos1.reshape(NW, TPW), pos2.reshape(NW, TPW)], axis=0)
    w_all = jnp.concatenate(
        [w1n.reshape(NW, TPW), w2n.reshape(NW, TPW)], axis=0)
    xs = _get_dispatch()(x2, pos_all)
    ys = _ffn(arr_e.reshape(MAX_TILES), valid.reshape(MAX_TILES),
              xblk.reshape(MAX_TILES), xs, W1, W3, W2)
    out2 = _get_combine()(ys, pos_all, w_all)
    return out2.reshape(x.shape), aux.reshape(())


# speedup vs baseline: 2.2998x; 1.2998x over previous
import functools

import jax
import jax.numpy as jnp
from jax import lax
from jax.experimental import pallas as pl
from jax.experimental.pallas import tpu as pltpu
from jax.experimental.pallas import tpu_sc as plsc

D_MODEL = 768
D_FFN = 3072
N_EXPERTS = 8
T = 2048
F_BLK = 768
N_FBLK = D_FFN // F_BLK
ROWS = 256
MAX_TILES = 24
P = MAX_TILES * ROWS
NW = 32
TPW = T // NW


def _router_body(x_ref, wg_ref, pos1_ref, pos2_ref, w1n_ref, w2n_ref,
                 arre_ref, valid_ref, xblk_ref, aux_ref):
    xb = x_ref[...]
    logits = jnp.dot(xb, wg_ref[...], preferred_element_type=jnp.float32)
    mu = jnp.mean(logits, axis=1, keepdims=True)
    var = jnp.sum((logits - mu) ** 2, axis=1, keepdims=True) / (N_EXPERTS - 1)
    aux_ref[...] = jnp.mean(var).reshape(1, 1)

    m1 = jnp.max(logits, axis=1, keepdims=True)
    p = jnp.exp(logits - m1)
    probs = p / jnp.sum(p, axis=1, keepdims=True)
    iota8 = lax.broadcasted_iota(jnp.int32, (T, N_EXPERTS), 1)
    w1v = jnp.max(probs, axis=1, keepdims=True)
    e1 = jnp.min(jnp.where(probs == w1v, iota8, N_EXPERTS), axis=1, keepdims=True)
    oh1 = iota8 == e1
    pm = jnp.where(oh1, -1e30, probs)
    w2v = jnp.max(pm, axis=1, keepdims=True)
    e2 = jnp.min(jnp.where(pm == w2v, iota8, N_EXPERTS), axis=1, keepdims=True)
    oh2 = iota8 == e2
    wsum = w1v + w2v
    w1n_ref[...] = w1v / wsum
    w2n_ref[...] = w2v / wsum

    mask = oh1.astype(jnp.float32) + oh2.astype(jnp.float32)
    ir = lax.broadcasted_iota(jnp.int32, (T, T), 0)
    ic = lax.broadcasted_iota(jnp.int32, (T, T), 1)
    lower = (ir > ic).astype(jnp.float32)
    rank = jnp.dot(lower, mask, preferred_element_type=jnp.float32)

    cnt = jnp.sum(mask, axis=0, keepdims=True)
    padded = jnp.ceil(cnt / ROWS) * ROWS
    k8 = lax.broadcasted_iota(jnp.int32, (N_EXPERTS, N_EXPERTS), 0)
    e8 = lax.broadcasted_iota(jnp.int32, (N_EXPERTS, N_EXPERTS), 1)
    lt8 = (k8 < e8).astype(jnp.float32)
    offs = jnp.dot(padded, lt8, preferred_element_type=jnp.float32)
    n_act = jnp.sum(padded) / ROWS

    pos1_ref[...] = (jnp.sum(jnp.where(oh1, rank + offs, 0.0), axis=1,
                             keepdims=True)).astype(jnp.int32)
    pos2_ref[...] = (jnp.sum(jnp.where(oh2, rank + offs, 0.0), axis=1,
                             keepdims=True)).astype(jnp.int32)

    it = lax.broadcasted_iota(jnp.int32, (MAX_TILES, N_EXPERTS), 0).astype(jnp.float32)
    ie = lax.broadcasted_iota(jnp.int32, (MAX_TILES, N_EXPERTS), 1).astype(jnp.float32)
    i_c = jnp.minimum(it, n_act - 1.0)
    start = offs / ROWS
    ntil = padded / ROWS
    ind = ((i_c >= start) & (i_c < start + ntil)).astype(jnp.float32)
    arre_ref[...] = jnp.sum(ie * ind, axis=1, keepdims=True).astype(jnp.int32)
    it1 = lax.broadcasted_iota(jnp.int32, (MAX_TILES, 1), 0).astype(jnp.float32)
    valid_ref[...] = (it1 < n_act).astype(jnp.int32)
    xblk_ref[...] = jnp.minimum(it1, n_act - 1.0).astype(jnp.int32)


def _router(x2, Wg):
    outs = pl.pallas_call(
        _router_body,
        out_shape=[
            jax.ShapeDtypeStruct((T, 1), jnp.int32),
            jax.ShapeDtypeStruct((T, 1), jnp.int32),
            jax.ShapeDtypeStruct((T, 1), jnp.float32),
            jax.ShapeDtypeStruct((T, 1), jnp.float32),
            jax.ShapeDtypeStruct((MAX_TILES, 1), jnp.int32),
            jax.ShapeDtypeStruct((MAX_TILES, 1), jnp.int32),
            jax.ShapeDtypeStruct((MAX_TILES, 1), jnp.int32),
            jax.ShapeDtypeStruct((1, 1), jnp.float32),
        ],
    )(x2, Wg)
    return outs


@functools.cache
def _get_dispatch():
    mesh = plsc.VectorSubcoreMesh(core_axis_name="c", subcore_axis_name="s")

    @functools.partial(
        pl.kernel,
        out_type=jax.ShapeDtypeStruct((P, D_MODEL), jnp.float32),
        mesh=mesh,
        scratch_types=[
            pltpu.VMEM((TPW, D_MODEL), jnp.float32),
            pltpu.VMEM((TPW,), jnp.int32),
            pltpu.VMEM((TPW,), jnp.int32),
            pltpu.SemaphoreType.DMA,
        ],
    )
    def _dispatch(x_hbm, pos_hbm, xs_hbm, buf, idx1, idx2, sem):
        wid = lax.axis_index("s") * 2 + lax.axis_index("c")
        base = wid * TPW
        pltpu.sync_copy(x_hbm.at[pl.ds(base, TPW)], buf)
        pltpu.sync_copy(pos_hbm.at[wid], idx1)
        pltpu.sync_copy(pos_hbm.at[NW + wid], idx2)
        pltpu.async_copy(buf, xs_hbm.at[idx1], sem).wait()
        pltpu.async_copy(buf, xs_hbm.at[idx2], sem).wait()

    return _dispatch


def _ffn_body(se, sv, sx, x_ref, w1_ref, w3_ref, w2_ref, y_ref):
    i = pl.program_id(0)

    @pl.when(sv[i] == 1)
    def _():
        xb = x_ref[...]
        a = jnp.dot(xb, w1_ref[0], preferred_element_type=jnp.float32)
        b = jnp.dot(xb, w3_ref[0], preferred_element_type=jnp.float32)
        h = (a * jax.nn.sigmoid(a)) * b
        y_ref[...] = jnp.dot(h, w2_ref[0], preferred_element_type=jnp.float32)


def _ffn(arr_e, valid, xblk, xs, W1, W3, W2):
    grid_spec = pltpu.PrefetchScalarGridSpec(
        num_scalar_prefetch=3,
        grid=(MAX_TILES,),
        in_specs=[
            pl.BlockSpec((ROWS, D_MODEL), lambda i, se, sv, sx: (sx[i], 0)),
            pl.BlockSpec((1, D_MODEL, D_FFN), lambda i, se, sv, sx: (se[i], 0, 0)),
            pl.BlockSpec((1, D_MODEL, D_FFN), lambda i, se, sv, sx: (se[i], 0, 0)),
            pl.BlockSpec((1, D_FFN, D_MODEL), lambda i, se, sv, sx: (se[i], 0, 0)),
        ],
        out_specs=pl.BlockSpec((ROWS, D_MODEL), lambda i, se, sv, sx: (sx[i], 0)),
    )
    return pl.pallas_call(
        _ffn_body,
        grid_spec=grid_spec,
        out_shape=jax.ShapeDtypeStruct((P, D_MODEL), jnp.float32),
        compiler_params=pltpu.CompilerParams(
            vmem_limit_bytes=100 * 1024 * 1024),
    )(arr_e, valid, xblk, xs, W1, W3, W2)


@functools.cache
def _get_combine():
    mesh = plsc.VectorSubcoreMesh(core_axis_name="c", subcore_axis_name="s")

    @functools.partial(
        pl.kernel,
        out_type=jax.ShapeDtypeStruct((T, D_MODEL), jnp.float32),
        mesh=mesh,
        scratch_types=[
            pltpu.VMEM((TPW, D_MODEL), jnp.float32),
            pltpu.VMEM((TPW, D_MODEL), jnp.float32),
            pltpu.VMEM((TPW,), jnp.int32),
            pltpu.VMEM((TPW,), jnp.int32),
            pltpu.VMEM((TPW,), jnp.float32),
            pltpu.VMEM((TPW,), jnp.float32),
            pltpu.SemaphoreType.DMA,
        ],
    )
    def _combine(y_hbm, pos_hbm, w_hbm, out_hbm, b1, b2, idx1, idx2, wv1, wv2, sem):
        wid = lax.axis_index("s") * 2 + lax.axis_index("c")
        base = wid * TPW
        pltpu.sync_copy(pos_hbm.at[wid], idx1)
        pltpu.sync_copy(pos_hbm.at[NW + wid], idx2)
        pltpu.sync_copy(w_hbm.at[wid], wv1)
        pltpu.sync_copy(w_hbm.at[NW + wid], wv2)
        pltpu.async_copy(y_hbm.at[idx1], b1, sem).wait()
        pltpu.async_copy(y_hbm.at[idx2], b2, sem).wait()

        def body(g, carry):
            wvec1 = wv1[pl.ds(g * 16, 16)]
            wvec2 = wv2[pl.ds(g * 16, 16)]
            for lane in range(16):
                t = g * 16 + lane
                ws1 = wvec1[lane]
                ws2 = wvec2[lane]
                for j in range(D_MODEL // 16):
                    s = pl.ds(j * 16, 16)
                    b1[t, s] = ws1 * b1[t, s] + ws2 * b2[t, s]
            return carry

        lax.fori_loop(0, TPW // 16, body, 0)
        pltpu.sync_copy(b1, out_hbm.at[pl.ds(base, TPW)])

    return _combine


def kernel(x, Wg, W1, W2, W3):
    x2 = x.reshape(T, D_MODEL)
    pos1, pos2, w1n, w2n, arr_e, valid, xblk, aux = _router(x2, Wg)
    pos_all = jnp.concatenate(
        [pos1.reshape(NW, TPW), pos2.reshape(NW, TPW)], axis=0)
    w_all = jnp.concatenate(
        [w1n.reshape(NW, TPW), w2n.reshape(NW, TPW)], axis=0)
    xs = _get_dispatch()(x2, pos_all)
    ys = _ffn(arr_e.reshape(MAX_TILES), valid.reshape(MAX_TILES),
              xblk.reshape(MAX_TILES), xs, W1, W3, W2)
    out2 = _get_combine()(ys, pos_all, w_all)
    return out2.reshape(x.shape), aux.reshape(())
